# baseline (speedup 1.0000x reference)
import functools

import jax
import jax.numpy as jnp
from jax import lax
from jax.experimental import pallas as pl
from jax.experimental.pallas import tpu as pltpu
from jax.experimental.pallas import tpu_sc as plsc

_BATCH = 16384
_EMBED_DIM = 64
_HIDDEN = 128
_OUT = 1000
_NUM_ROWS = _BATCH * 2
_NC, _NS = 2, 16
_NW = _NC * _NS
_B_PER_W = _BATCH // _NW
_R_PER_W = 2 * _B_PER_W
_LANES = 16
_WPR = _EMBED_DIM // 2


@functools.cache
def _make_sc_gather():
  mesh = plsc.VectorSubcoreMesh(core_axis_name="c", subcore_axis_name="s")

  @functools.partial(
      pl.kernel,
      out_type=jax.ShapeDtypeStruct((_BATCH, 2 * _WPR), jnp.int32),
      mesh=mesh,
      scratch_types=[
          pltpu.VMEM((_R_PER_W // 128, 128), jnp.int32),
          pltpu.VMEM((_B_PER_W // 2, 128), jnp.int32),
          pltpu.VMEM((_B_PER_W, 2 * _WPR), jnp.int32),
          pltpu.SemaphoreType.DMA,
      ],
  )
  def gather_kernel(idx_hbm, table_hbm, out_hbm, idx_v, lines_v, rows_v, sem):
    wid = lax.axis_index("s") * _NC + lax.axis_index("c")
    base_b = wid * _B_PER_W
    nrow = _R_PER_W // 128
    pltpu.sync_copy(idx_hbm.at[pl.ds(wid * nrow, nrow)], idx_v)

    for quarter in range(4):
      rows_q = range(quarter * nrow // 4, (quarter + 1) * nrow // 4)
      for row in rows_q:
        def chunk(cc, carry, row=row):
          v = idx_v[row, pl.ds(cc * _LANES, _LANES)]
          for lane in range(_LANES):
            r_idx = v[lane]
            line = (row % (nrow // 4)) * 128 + cc * _LANES + lane
            pltpu.async_copy(
                table_hbm.at[r_idx // 32, (r_idx // 4) % 8, :],
                lines_v.at[line, :],
                sem,
            )
          return carry

        lax.fori_loop(0, 128 // _LANES, chunk, 0)
      pltpu.make_async_copy(idx_hbm.at[pl.ds(0, 256)], lines_v, sem).wait()
      for row in rows_q:
        def extract(cc, carry, row=row):
          v = idx_v[row, pl.ds(cc * _LANES, _LANES)]
          for lane in range(_LANES):
            r_idx = v[lane]
            line = (row % (nrow // 4)) * 128 + cc * _LANES + lane
            off = (r_idx % 4) * _WPR
            b_local = row * 64 + cc * (_LANES // 2) + lane // 2
            col = _WPR * (lane % 2)
            rows_v[b_local, pl.ds(col, _LANES)] = (
                lines_v[line, pl.ds(off, _LANES)])
            rows_v[b_local, pl.ds(col + _LANES, _LANES)] = (
                lines_v[line, pl.ds(off + _LANES, _LANES)])
          return carry

        lax.fori_loop(0, 128 // _LANES, extract, 0)
    pltpu.sync_copy(rows_v, out_hbm.at[pl.ds(base_b, _B_PER_W)])

  return gather_kernel


def _mlp_body(e_ref, w1e_ref, w1o_ref, b1_ref, w2_ref, b2_ref, outt_ref):
  w = e_ref[...]
  lo = lax.bitcast_convert_type(w << 16, jnp.float32).astype(jnp.bfloat16)
  hi = lax.bitcast_convert_type(
      w & jnp.int32(-65536), jnp.float32).astype(jnp.bfloat16)
  h = lax.dot_general(lo, w1e_ref[...], (((1,), (1,)), ((), ())),
                      preferred_element_type=jnp.float32)
  h = h + lax.dot_general(hi, w1o_ref[...], (((1,), (1,)), ((), ())),
                          preferred_element_type=jnp.float32)
  h = jnp.maximum(h + b1_ref[...], 0.0).astype(jnp.bfloat16)
  out = lax.dot_general(w2_ref[...], h, (((1,), (1,)), ((), ())),
                        preferred_element_type=jnp.float32)
  out = out + b2_ref[...]
  m = jnp.max(out, axis=0, keepdims=True)
  s = out - m
  lse = jnp.log(jnp.sum(jnp.exp(s), axis=0, keepdims=True))
  outt_ref[...] = s - lse


def _make_mlp(bb: int):
  grid = (_BATCH // bb,)
  return pl.pallas_call(
      _mlp_body,
      grid=grid,
      in_specs=[
          pl.BlockSpec((bb, 2 * _WPR), lambda i: (i, 0)),
          pl.BlockSpec((_HIDDEN, 2 * _WPR), lambda i: (0, 0)),
          pl.BlockSpec((_HIDDEN, 2 * _WPR), lambda i: (0, 0)),
          pl.BlockSpec((1, _HIDDEN), lambda i: (0, 0)),
          pl.BlockSpec((_OUT, _HIDDEN), lambda i: (0, 0)),
          pl.BlockSpec((_OUT, 1), lambda i: (0, 0)),
      ],
      out_specs=pl.BlockSpec((_OUT, bb), lambda i: (0, i)),
      out_shape=jax.ShapeDtypeStruct((_OUT, _BATCH), jnp.float32),
  )


_mlp = _make_mlp(512)


def kernel(x, embed, W1, b1, W2, b2):
  tabw = lax.bitcast_convert_type(
      embed.astype(jnp.bfloat16).reshape(31250, 8, 128, 2), jnp.int32)
  e = _make_sc_gather()(x.reshape(_NUM_ROWS // 128, 128), tabw)
  w1b = W1.astype(jnp.bfloat16)
  out_t = _mlp(e, w1b[:, 0::2], w1b[:, 1::2], b1.reshape(1, _HIDDEN),
               W2.astype(jnp.bfloat16), b2.reshape(_OUT, 1))
  return out_t.T


# speedup vs baseline: 73.4809x; 73.4809x over previous
import functools

import jax
import jax.numpy as jnp
from jax import lax
from jax.experimental import pallas as pl
from jax.experimental.pallas import tpu as pltpu
from jax.experimental.pallas import tpu_sc as plsc

_BATCH = 16384
_EMBED_DIM = 64
_HIDDEN = 128
_OUT = 1000
_NUM_ROWS = _BATCH * 2
_NC, _NS = 2, 16
_NW = _NC * _NS
_B_PER_W = _BATCH // _NW
_R_PER_W = 2 * _B_PER_W
_LANES = 16


@functools.cache
def _make_sc_gather():
  mesh = plsc.VectorSubcoreMesh(core_axis_name="c", subcore_axis_name="s")

  @functools.partial(
      pl.kernel,
      out_type=jax.ShapeDtypeStruct((_BATCH, 2 * _EMBED_DIM), jnp.float32),
      mesh=mesh,
      scratch_types=[
          pltpu.VMEM((_R_PER_W // 128, 128), jnp.int32),
          pltpu.VMEM((_B_PER_W, 2 * _EMBED_DIM), jnp.float32),
          pltpu.SemaphoreType.DMA,
      ],
  )
  def gather_kernel(idx_hbm, table_hbm, out_hbm, idx_v, rows_v, sem):
    wid = lax.axis_index("s") * _NC + lax.axis_index("c")
    base_b = wid * _B_PER_W
    nrow = _R_PER_W // 128
    pltpu.sync_copy(idx_hbm.at[pl.ds(wid * nrow, nrow)], idx_v)

    for row in range(nrow):
      def chunk(cc, carry, row=row):
        v = idx_v[row, pl.ds(cc * _LANES, _LANES)]
        for lane in range(_LANES):
          r_idx = v[lane]
          b_local = row * 64 + cc * (_LANES // 2) + lane // 2
          col = _EMBED_DIM * (lane % 2)
          pltpu.async_copy(
              table_hbm.at[r_idx // 8, r_idx % 8, :],
              rows_v.at[b_local, pl.ds(col, _EMBED_DIM)],
              sem,
          )
        return carry

      lax.fori_loop(0, 128 // _LANES, chunk, 0)
    pltpu.make_async_copy(
        out_hbm.at[pl.ds(base_b, _B_PER_W)], rows_v, sem).wait()
    pltpu.sync_copy(rows_v, out_hbm.at[pl.ds(base_b, _B_PER_W)])

  return gather_kernel


def _mlp_body(e_ref, w1_ref, b1_ref, w2_ref, b2_ref, outt_ref):
  e = e_ref[...].astype(jnp.bfloat16)
  h = lax.dot_general(e, w1_ref[...], (((1,), (1,)), ((), ())),
                      preferred_element_type=jnp.float32)
  h = jnp.maximum(h + b1_ref[...], 0.0).astype(jnp.bfloat16)
  out = lax.dot_general(w2_ref[...], h, (((1,), (1,)), ((), ())),
                        preferred_element_type=jnp.float32)
  out = out + b2_ref[...]
  m = jnp.max(out, axis=0, keepdims=True)
  s = out - m
  lse = jnp.log(jnp.sum(jnp.exp(s), axis=0, keepdims=True))
  outt_ref[...] = s - lse


def _make_mlp(bb: int):
  grid = (_BATCH // bb,)
  return pl.pallas_call(
      _mlp_body,
      grid=grid,
      in_specs=[
          pl.BlockSpec((bb, 2 * _EMBED_DIM), lambda i: (i, 0)),
          pl.BlockSpec((_HIDDEN, 2 * _EMBED_DIM), lambda i: (0, 0)),
          pl.BlockSpec((1, _HIDDEN), lambda i: (0, 0)),
          pl.BlockSpec((_OUT, _HIDDEN), lambda i: (0, 0)),
          pl.BlockSpec((_OUT, 1), lambda i: (0, 0)),
      ],
      out_specs=pl.BlockSpec((_OUT, bb), lambda i: (0, i)),
      out_shape=jax.ShapeDtypeStruct((_OUT, _BATCH), jnp.float32),
  )


_mlp = _make_mlp(512)


def kernel(x, embed, W1, b1, W2, b2):
  e = _make_sc_gather()(x.reshape(_NUM_ROWS // 128, 128),
                        embed.reshape(125000, 8, _EMBED_DIM))
  out_t = _mlp(e, W1.astype(jnp.bfloat16), b1.reshape(1, _HIDDEN),
               W2.astype(jnp.bfloat16), b2.reshape(_OUT, 1))
  return out_t.T


# speedup vs baseline: 75.5866x; 1.0287x over previous
import functools

import jax
import jax.numpy as jnp
from jax import lax
from jax.experimental import pallas as pl
from jax.experimental.pallas import tpu as pltpu
from jax.experimental.pallas import tpu_sc as plsc

_BATCH = 16384
_EMBED_DIM = 64
_HIDDEN = 128
_OUT = 1000
_NUM_ROWS = _BATCH * 2
_NC, _NS = 2, 16
_NW = _NC * _NS
_B_PER_W = _BATCH // _NW
_R_PER_W = 2 * _B_PER_W
_LANES = 16


@functools.cache
def _make_sc_gather():
  mesh = plsc.VectorSubcoreMesh(core_axis_name="c", subcore_axis_name="s")

  @functools.partial(
      pl.kernel,
      out_type=jax.ShapeDtypeStruct((_BATCH, 2 * _EMBED_DIM), jnp.float32),
      mesh=mesh,
      scratch_types=[
          pltpu.VMEM((_R_PER_W // 128, 128), jnp.int32),
          pltpu.VMEM((_B_PER_W, 2 * _EMBED_DIM), jnp.float32),
          pltpu.SemaphoreType.DMA,
      ],
  )
  def gather_kernel(idx_hbm, table_hbm, out_hbm, idx_v, rows_v, sem):
    wid = lax.axis_index("s") * _NC + lax.axis_index("c")
    base_b = wid * _B_PER_W
    nrow = _R_PER_W // 128
    pltpu.sync_copy(idx_hbm.at[pl.ds(wid * nrow, nrow)], idx_v)

    for row in range(nrow):
      def chunk(cc, carry, row=row):
        v = idx_v[row, pl.ds(cc * _LANES, _LANES)]
        for lane in range(_LANES):
          r_idx = v[lane]
          b_local = row * 64 + cc * (_LANES // 2) + lane // 2
          col = _EMBED_DIM * (lane % 2)
          pltpu.async_copy(
              table_hbm.at[r_idx // 8, r_idx % 8, :],
              rows_v.at[b_local, pl.ds(col, _EMBED_DIM)],
              sem,
          )
        return carry

      lax.fori_loop(0, 128 // _LANES, chunk, 0)
    pltpu.make_async_copy(
        out_hbm.at[pl.ds(base_b, _B_PER_W)], rows_v, sem).wait()
    pltpu.sync_copy(rows_v, out_hbm.at[pl.ds(base_b, _B_PER_W)])

  return gather_kernel


def _mlp_body(e_ref, w1_ref, b1_ref, w2_ref, b2_ref, outt_ref):
  e = e_ref[...].astype(jnp.bfloat16)
  h = lax.dot_general(e, w1_ref[...], (((1,), (1,)), ((), ())),
                      preferred_element_type=jnp.float32)
  h = jnp.maximum(h + b1_ref[...], 0.0).astype(jnp.bfloat16)
  out = lax.dot_general(w2_ref[...], h, (((1,), (1,)), ((), ())),
                        preferred_element_type=jnp.float32)
  out = out + b2_ref[...]
  m = jnp.max(out, axis=0, keepdims=True)
  s = out - m
  lse = jnp.log(jnp.sum(jnp.exp(s), axis=0, keepdims=True))
  outt_ref[...] = s - lse


def _make_mlp(bb: int):
  grid = (_BATCH // bb,)
  return pl.pallas_call(
      _mlp_body,
      grid=grid,
      in_specs=[
          pl.BlockSpec((bb, 2 * _EMBED_DIM), lambda i: (i, 0)),
          pl.BlockSpec((_HIDDEN, 2 * _EMBED_DIM), lambda i: (0, 0)),
          pl.BlockSpec((1, _HIDDEN), lambda i: (0, 0)),
          pl.BlockSpec((_OUT, _HIDDEN), lambda i: (0, 0)),
          pl.BlockSpec((_OUT, 1), lambda i: (0, 0)),
      ],
      out_specs=pl.BlockSpec((_OUT, bb), lambda i: (0, i)),
      out_shape=jax.ShapeDtypeStruct((_OUT, _BATCH), jnp.float32),
  )


_mlp = _make_mlp(1024)


def kernel(x, embed, W1, b1, W2, b2):
  e = _make_sc_gather()(x.reshape(_NUM_ROWS // 128, 128),
                        embed.reshape(125000, 8, _EMBED_DIM))
  out_t = _mlp(e, W1.astype(jnp.bfloat16), b1.reshape(1, _HIDDEN),
               W2.astype(jnp.bfloat16), b2.reshape(_OUT, 1))
  return out_t.T
